# Initial kernel scaffold; baseline (speedup 1.0000x reference)
import jax
import jax.numpy as jnp
from jax.experimental import pallas as pl


def kernel(x, params, edge_index, graph_ids):
    raise NotImplementedError("write your pallas kernel here")


# speedup vs baseline: 3.5082x; 3.5082x over previous
import functools

import jax
import jax.numpy as jnp
from jax import lax
from jax.experimental import pallas as pl
from jax.experimental.pallas import tpu as pltpu
from jax.experimental.pallas import tpu_sc as plsc

N = 10000
D = 128
E = 320000
G = 16
OUT = 64
NUM_LAYERS = 5
BN_EPS = 1e-5

NC = 2
NS = 16
NW = NC * NS
K = 128
CH = 79
EP = NW * K * CH
NP = 10112
RPS = NP // NS

_mesh = plsc.VectorSubcoreMesh(core_axis_name="c", subcore_axis_name="s")


@functools.partial(
    pl.kernel,
    mesh=_mesh,
    out_type=jax.ShapeDtypeStruct((NC, NP, D), jnp.float32),
    scratch_types=[
        pltpu.VMEM((K,), jnp.int32),
        pltpu.VMEM((K,), jnp.int32),
        pltpu.VMEM((K, D), jnp.float32),
        pltpu.VMEM_SHARED((NP, D), jnp.float32),
        pltpu.SemaphoreType.DMA,
    ],
)
def _agg(h_hbm, src_hbm, dst_hbm, out_hbm, idxd, idxs, rows, pooled, sem):
    cid = lax.axis_index("c")
    sid = lax.axis_index("s")
    wid = cid * NS + sid
    zero = jnp.zeros((16,), jnp.float32)

    @pl.loop(0, K)
    def _(r):
        @pl.loop(0, D, step=16)
        def _(cc):
            rows[r, pl.ds(cc, 16)] = zero

    @pl.loop(0, RPS - K, step=K)
    def _(r0):
        pltpu.sync_copy(rows, pooled.at[pl.ds(sid * RPS + r0, K), :])

    pltpu.sync_copy(rows.at[pl.ds(0, RPS % K), :],
                    pooled.at[pl.ds(sid * RPS + (RPS - RPS % K), RPS % K), :])
    plsc.subcore_barrier()

    @pl.loop(0, CH)
    def _(ch):
        base = (wid * CH + ch) * K
        pltpu.sync_copy(dst_hbm.at[pl.ds(base, K)], idxd)
        pltpu.async_copy(h_hbm.at[idxd], rows, sem).wait()
        pltpu.sync_copy(src_hbm.at[pl.ds(base, K)], idxs)
        pltpu.sync_copy(rows, pooled.at[idxs], add=True)

    plsc.subcore_barrier()
    pltpu.sync_copy(pooled.at[pl.ds(sid * RPS, RPS), :],
                    out_hbm.at[cid, pl.ds(sid * RPS, RPS), :])


def _bn_relu(h, g, b):
    m = jnp.mean(h, axis=0, keepdims=True)
    d = h - m
    v = jnp.mean(d * d, axis=0, keepdims=True)
    return jnp.maximum(d * lax.rsqrt(v + BN_EPS) * g + b, 0.0)


def _mlp_body(eps_ref, p0_ref, p1_ref, h_ref, w1_ref, b1_ref, g1_ref, bb1_ref,
              w2_ref, b2_ref, g2_ref, bb2_ref, out_ref):
    pooled = p0_ref[...] + p1_ref[...] + (1.0 + eps_ref[0, 0]) * h_ref[...]
    h1 = jnp.dot(pooled, w1_ref[...], preferred_element_type=jnp.float32)
    h1 = _bn_relu(h1 + b1_ref[...], g1_ref[...], bb1_ref[...])
    h2 = jnp.dot(h1, w2_ref[...], preferred_element_type=jnp.float32)
    out_ref[...] = _bn_relu(h2 + b2_ref[...], g2_ref[...], bb2_ref[...])


_mlp = pl.pallas_call(
    _mlp_body,
    out_shape=jax.ShapeDtypeStruct((N, D), jnp.float32),
)


def _pool_body(gid_ref, h0_ref, h1_ref, h2_ref, h3_ref, h4_ref,
               w0_ref, w1_ref, w2_ref, w3_ref, w4_ref,
               b0_ref, b1_ref, b2_ref, b3_ref, b4_ref, out_ref):
    onehot = (lax.broadcasted_iota(jnp.int32, (G, N), 0)
              == gid_ref[...]).astype(jnp.float32)
    score = jnp.zeros((G, OUT), jnp.float32)
    for h_ref, w_ref, b_ref in ((h0_ref, w0_ref, b0_ref),
                                (h1_ref, w1_ref, b1_ref),
                                (h2_ref, w2_ref, b2_ref),
                                (h3_ref, w3_ref, b3_ref),
                                (h4_ref, w4_ref, b4_ref)):
        ph = jnp.dot(onehot, h_ref[...], preferred_element_type=jnp.float32)
        score = score + jnp.dot(ph, w_ref[...],
                                preferred_element_type=jnp.float32) + b_ref[...]
    out_ref[...] = score


_pool = pl.pallas_call(
    _pool_body,
    out_shape=jax.ShapeDtypeStruct((G, OUT), jnp.float32),
)


def kernel(x, params, edge_index, graph_ids):
    src = edge_index[0]
    dst = edge_index[1]
    src_p = jnp.concatenate([src, jnp.full((EP - E,), NP - 1, jnp.int32)])
    dst_p = jnp.concatenate([dst, jnp.zeros((EP - E,), jnp.int32)])
    h = x
    hidden = [x]
    for l in range(NUM_LAYERS - 1):
        parts = _agg(h, src_p, dst_p)
        eps = params['eps'][l].reshape(1, 1)
        h = _mlp(eps, parts[0, :N, :], parts[1, :N, :], h,
                 params[f'mlp{l}_W1'], params[f'mlp{l}_b1'].reshape(1, -1),
                 params[f'mlp{l}_bn_g'].reshape(1, -1),
                 params[f'mlp{l}_bn_b'].reshape(1, -1),
                 params[f'mlp{l}_W2'], params[f'mlp{l}_b2'].reshape(1, -1),
                 params[f'bn{l}_g'].reshape(1, -1),
                 params[f'bn{l}_b'].reshape(1, -1))
        hidden.append(h)
    gid = graph_ids.reshape(1, N)
    args = ([gid] + hidden
            + [params[f'pred{l}_W'] for l in range(NUM_LAYERS)]
            + [params[f'pred{l}_b'].reshape(1, -1) for l in range(NUM_LAYERS)])
    return _pool(*args)
